# baseline (speedup 1.0000x reference)
import functools

import jax
import jax.numpy as jnp
from jax import lax
from jax.experimental import pallas as pl
from jax.experimental.pallas import tpu as pltpu
from jax.experimental.pallas import tpu_sc as plsc

K = 128
EDGE_TYPES = 16384
B, N = 4, 256
E = B * N * N
ROWS = E // 128

_NC = 2
_NS = 16
_L = 16
_NW = _NC * _NS
_EPW = E // _NW

_RB = 256
_LOG2E = 1.4426950408889634


def _sc_fma_body(et_hbm, x_hbm, tab_hbm, out_hbm,
                 idx_v, x_v, xx_v, tab_v, sem):
    wid = lax.axis_index("s") * _NC + lax.axis_index("c")
    base = wid * _EPW
    c1 = pltpu.async_copy(et_hbm.at[pl.ds(base, _EPW)], idx_v, sem)
    c2 = pltpu.async_copy(x_hbm.at[pl.ds(base, _EPW)], x_v, sem)
    c3 = pltpu.async_copy(tab_hbm, tab_v, sem)
    c1.wait()
    c2.wait()
    c3.wait()

    @plsc.parallel_loop(0, _EPW // _L, unroll=16)
    def _(i):
        s = pl.ds(i * _L, _L)
        idx = idx_v[s]
        m = plsc.load_gather(tab_v, [idx])
        bb = plsc.load_gather(tab_v, [idx + EDGE_TYPES])
        xx_v[s] = m * x_v[s] + bb

    pltpu.sync_copy(xx_v, out_hbm.at[pl.ds(base, _EPW)])


@functools.cache
def _sc_fma():
    return pl.kernel(
        _sc_fma_body,
        mesh=plsc.VectorSubcoreMesh(core_axis_name="c", subcore_axis_name="s"),
        compiler_params=pltpu.CompilerParams(needs_layout_passes=False),
        out_type=jax.ShapeDtypeStruct((E,), jnp.float32),
        scratch_types=[
            pltpu.VMEM((_EPW,), jnp.int32),
            pltpu.VMEM((_EPW,), jnp.float32),
            pltpu.VMEM((_EPW,), jnp.float32),
            pltpu.VMEM((2 * EDGE_TYPES,), jnp.float32),
            pltpu.SemaphoreType.DMA,
        ],
    )


def _tc_rbf_body(mean_ref, temp_ref, xx_ref, out_ref):
    mean = mean_ref[0]
    ntemp = -jnp.abs(temp_ref[0]) * _LOG2E
    xx = xx_ref[...]
    d = xx[:, :, None] - mean[None, None, :]
    out_ref[...] = jnp.exp2(d * d * ntemp[None, None, :])


def _tc_rbf(xx2, meanr, tempr):
    return pl.pallas_call(
        _tc_rbf_body,
        grid=(ROWS // _RB,),
        in_specs=[
            pl.BlockSpec((1, K), lambda i: (0, 0)),
            pl.BlockSpec((1, K), lambda i: (0, 0)),
            pl.BlockSpec((_RB, 128), lambda i: (i, 0)),
        ],
        out_specs=pl.BlockSpec((_RB, 128, K), lambda i: (i, 0, 0)),
        out_shape=jax.ShapeDtypeStruct((ROWS, 128, K), jnp.float32),
    )(meanr, tempr, xx2)


def kernel(x, edge_types, means, temps, mul_w, bias_w):
    et = edge_types.reshape(E).astype(jnp.int32)
    xf = x.reshape(E).astype(jnp.float32)
    tab = jnp.concatenate([mul_w.reshape(EDGE_TYPES),
                           bias_w.reshape(EDGE_TYPES)])
    xx = _sc_fma()(et, xf, tab)
    out = _tc_rbf(xx.reshape(ROWS, 128),
                  means.reshape(1, K), temps.reshape(1, K))
    return out.reshape(B, N, N, K).astype(means.dtype)


# speedup vs baseline: 1.0667x; 1.0667x over previous
import functools

import jax
import jax.numpy as jnp
from jax import lax
from jax.experimental import pallas as pl
from jax.experimental.pallas import tpu as pltpu
from jax.experimental.pallas import tpu_sc as plsc

K = 128
EDGE_TYPES = 16384
B, N = 4, 256
E = B * N * N
ROWS = E // 128

_NC = 2
_NS = 16
_L = 16
_NW = _NC * _NS
_EPW = E // _NW

_LOG2E = 1.4426950408889634

_NQ = 8
_RB = 16
_NBUF = 2


def _sc_fma_body(et_hbm, x_hbm, mul_hbm, bias_hbm, out_hbm,
                 idx_v, x_v, xx_v, mul_v, bias_v, sem):
    wid = lax.axis_index("s") * _NC + lax.axis_index("c")
    base = wid * _EPW
    c1 = pltpu.async_copy(et_hbm.at[pl.ds(base, _EPW)], idx_v, sem)
    c2 = pltpu.async_copy(x_hbm.at[pl.ds(base, _EPW)], x_v, sem)
    c3 = pltpu.async_copy(mul_hbm, mul_v, sem)
    c4 = pltpu.async_copy(bias_hbm, bias_v, sem)
    c1.wait()
    c2.wait()
    c3.wait()
    c4.wait()

    @plsc.parallel_loop(0, _EPW // _L, unroll=8)
    def _(i):
        s = pl.ds(i * _L, _L)
        idx = idx_v[s]
        m = plsc.load_gather(mul_v, [idx])
        bb = plsc.load_gather(bias_v, [idx])
        xx_v[s] = m * x_v[s] + bb

    pltpu.sync_copy(xx_v, out_hbm.at[pl.ds(base, _EPW)])


@functools.cache
def _sc_fma():
    return pl.kernel(
        _sc_fma_body,
        mesh=plsc.VectorSubcoreMesh(core_axis_name="c", subcore_axis_name="s"),
        compiler_params=pltpu.CompilerParams(needs_layout_passes=False),
        out_type=jax.ShapeDtypeStruct((E,), jnp.float32),
        scratch_types=[
            pltpu.VMEM((_EPW,), jnp.int32),
            pltpu.VMEM((_EPW,), jnp.float32),
            pltpu.VMEM((_EPW,), jnp.float32),
            pltpu.VMEM((EDGE_TYPES,), jnp.float32),
            pltpu.VMEM((EDGE_TYPES,), jnp.float32),
            pltpu.SemaphoreType.DMA,
        ],
    )


def _tc_rbf(xx3, meanr, tempr):
    h = ROWS // _NQ
    nstep = h // _RB

    def body(mean_ref, temp_ref, xx_ref, out_ref, scr, sem):
        i = pl.program_id(0)
        mean = mean_ref[0]
        ntemp = -jnp.abs(temp_ref[0]) * _LOG2E
        buf = lax.rem(i, _NBUF)

        @pl.when(i >= _NBUF)
        def _():
            for q in range(_NQ):
                pltpu.make_async_copy(
                    scr.at[buf, q],
                    out_ref.at[q, pl.ds(0, _RB)],
                    sem.at[buf, q],
                ).wait()

        for q in range(_NQ):
            xx = xx_ref[q]
            d = xx[:, :, None] - mean[None, None, :]
            scr[buf, q] = jnp.exp2(d * d * ntemp[None, None, :])
            pltpu.async_copy(
                scr.at[buf, q],
                out_ref.at[q, pl.ds(i * _RB, _RB)],
                sem.at[buf, q],
            )

        @pl.when(i == nstep - 1)
        def _():
            for b in range(_NBUF):
                for q in range(_NQ):
                    pltpu.make_async_copy(
                        scr.at[b, q],
                        out_ref.at[q, pl.ds(0, _RB)],
                        sem.at[b, q],
                    ).wait()

    return pl.pallas_call(
        body,
        grid=(nstep,),
        in_specs=[
            pl.BlockSpec((1, K), lambda i: (0, 0)),
            pl.BlockSpec((1, K), lambda i: (0, 0)),
            pl.BlockSpec((_NQ, _RB, 128), lambda i: (0, i, 0)),
        ],
        out_specs=pl.BlockSpec(memory_space=pl.ANY),
        out_shape=jax.ShapeDtypeStruct((_NQ, h, 128, K), jnp.float32),
        scratch_shapes=[
            pltpu.VMEM((_NBUF, _NQ, _RB, 128, K), jnp.float32),
            pltpu.SemaphoreType.DMA((_NBUF, _NQ)),
        ],
    )(meanr, tempr, xx3)


def kernel(x, edge_types, means, temps, mul_w, bias_w):
    et = edge_types.reshape(E).astype(jnp.int32)
    xf = x.reshape(E).astype(jnp.float32)
    xx = _sc_fma()(et, xf, mul_w.reshape(EDGE_TYPES),
                   bias_w.reshape(EDGE_TYPES))
    out = _tc_rbf(xx.reshape(_NQ, ROWS // _NQ, 128),
                  means.reshape(1, K), temps.reshape(1, K))
    return out.reshape(B, N, N, K).astype(means.dtype)
